# baseline (speedup 1.0000x reference)
import functools

import jax
import jax.numpy as jnp
from jax import lax
from jax.experimental import pallas as pl
from jax.experimental.pallas import tpu as pltpu
from jax.experimental.pallas import tpu_sc as plsc

RES = 16
B = 64
N = 100000
F = RES ** 3
C = 40

NC = 2
NS = 16
NQ = 4

NPACK = 49920
NREM = N - 2 * NPACK
NT = NPACK // 128
CW = 26
CHC = CW * 128
NCHUNK = NT // CW
NV = CHC // 16

_mesh = plsc.VectorSubcoreMesh(
    core_axis_name="c", subcore_axis_name="s", num_cores=NC, num_subcores=NS
)


def _idx_body(x_ref, o_ref, o2_ref):
    xb = x_ref[...]
    mn = jnp.min(xb, axis=2, keepdims=True)
    mx = jnp.max(xb, axis=2, keepdims=True)
    rng = mx - mn
    rng = jnp.where(rng <= 0.0, jnp.ones_like(rng), rng)
    scl = float(RES) / rng
    t = jnp.minimum((xb - mn) * scl, 15.0).astype(jnp.int32)
    flat = (t[0] << 8) | (t[1] << 4) | t[2]
    o_ref[...] = (flat[:, :NPACK] << 16) | flat[:, NPACK:2 * NPACK]
    o2_ref[...] = jnp.concatenate(
        [flat[:, 2 * NPACK:], jnp.zeros((8, 256 - NREM), jnp.int32)], axis=1
    )


def _flat_indices(xt):
    return pl.pallas_call(
        _idx_body,
        grid=(B // 8,),
        in_specs=[pl.BlockSpec((3, 8, N), lambda g: (0, g, 0))],
        out_specs=[
            pl.BlockSpec((8, NPACK), lambda g: (g, 0)),
            pl.BlockSpec((8, 256), lambda g: (g, 0)),
        ],
        out_shape=[
            jax.ShapeDtypeStruct((B, NPACK), jnp.int32),
            jax.ShapeDtypeStruct((B, 256), jnp.int32),
        ],
    )(xt)


@functools.partial(
    pl.kernel,
    out_type=jax.ShapeDtypeStruct((NQ, B, F), jnp.float32),
    mesh=_mesh,
    scratch_types=[
        pltpu.VMEM((2, 8, CHC), jnp.int32),
        pltpu.VMEM((8 * F,), jnp.float32),
        pltpu.VMEM((8, F), jnp.float32),
        pltpu.VMEM((8, 256), jnp.int32),
        pltpu.SemaphoreType.DMA,
        pltpu.SemaphoreType.DMA,
    ],
    compiler_params=pltpu.CompilerParams(needs_layout_passes=False),
)
def _hist_sc(idx_ref, rem_ref, out_ref, bufs, hist1, hist2, rembuf,
             semA, semB):
    cid = lax.axis_index("c")
    sid = lax.axis_index("s")
    wid = cid * NS + sid
    g = wid // NQ
    q = wid % NQ
    nq = jnp.where(q == 3, 3, 4)
    row = pl.multiple_of(g * 8, 8)

    ones = jnp.ones((16,), jnp.float32)
    zeros = jnp.zeros((16,), jnp.float32)
    mask16 = jnp.full((16,), 0xFFFF, jnp.int32)

    @pl.loop(0, 8 * F // 128, unroll=8)
    def _zero(j):
        for u in range(8):
            hist1[pl.ds(j * 128 + u * 16, 16)] = zeros

    def start_fetch(c, slot, sem):
        col = pl.multiple_of(c * CHC, 128)
        pltpu.async_copy(
            idx_ref.at[pl.ds(row, 8), pl.ds(col, CHC)], bufs.at[slot], sem
        )

    def wait_fetch(slot, sem):
        pltpu.make_async_copy(
            idx_ref.at[pl.ds(0, 8), pl.ds(0, CHC)], bufs.at[slot], sem
        ).wait()

    def process(slot):
        @pl.loop(0, 8)
        def _leg(r):
            hrow = hist1.at[pl.ds(r * F, F)]

            def quad(j4, cc, hrow=hrow, r=r):
                base = j4 * 64
                ws = [bufs[slot, r, pl.ds(base + 16 * i, 16)]
                      for i in range(4)]
                flats = []
                for w in ws:
                    flats.append(w >> 16)
                    flats.append(w & mask16)
                for f_ in flats:
                    plsc.addupdate_scatter(hrow, [f_], ones)
                return cc

            lax.fori_loop(0, NV // 4, quad, 0, unroll=2)

    start_fetch(q, 0, semA)

    def step(k, carry):
        start_fetch(q + NQ * (2 * k + 1), 1, semB)
        wait_fetch(0, semA)
        process(0)

        @pl.when(2 * k + 2 < nq)
        def _():
            start_fetch(q + NQ * (2 * k + 2), 0, semA)

        wait_fetch(1, semB)
        process(1)
        return carry

    lax.fori_loop(0, nq // 2, step, 0)

    @pl.when(nq % 2 == 1)
    def _():
        wait_fetch(0, semA)
        process(0)

    @pl.when(q == 3)
    def _():
        pltpu.sync_copy(rem_ref.at[pl.ds(row, 8), :], rembuf)

        @pl.loop(0, 8)
        def _leg(r):
            hrow = hist1.at[pl.ds(r * F, F)]

            @pl.loop(0, NREM // 16)
            def _v(v, hrow=hrow, r=r):
                flat = rembuf[r, pl.ds(v * 16, 16)]
                plsc.addupdate_scatter(hrow, [flat], ones)

    for r in range(8):
        @pl.loop(0, F // 128, unroll=4)
        def _conv(jj, r=r):
            for u in range(8):
                s = jj * 128 + u * 16
                hist2[r, pl.ds(s, 16)] = hist1[pl.ds(r * F + s, 16)]

    pltpu.sync_copy(hist2, out_ref.at[q, pl.ds(row, 8), :])


def _mm_body(p_ref, w_ref, b_ref, o_ref):
    c = (p_ref[0] + p_ref[1] + p_ref[2] + p_ref[3]) * (1.0 / float(N))
    o_ref[...] = (
        lax.dot_general(
            c, w_ref[...], (((1,), (1,)), ((), ())),
            preferred_element_type=jnp.float32,
            precision=lax.Precision.HIGHEST,
        )
        + b_ref[...]
    )


def kernel(x, W, b):
    xt = jnp.transpose(x, (2, 0, 1))
    packed, rem = _flat_indices(xt)
    partials = _hist_sc(packed, rem)
    logits = pl.pallas_call(
        _mm_body,
        out_shape=jax.ShapeDtypeStruct((B, C), jnp.float32),
    )(partials, W, b.reshape(1, C))
    return logits
